# Initial kernel scaffold; baseline (speedup 1.0000x reference)
import jax
import jax.numpy as jnp
from jax.experimental import pallas as pl


def kernel(x, w1, w2):
    raise NotImplementedError("write your pallas kernel here")


# speedup vs baseline: 1.2529x; 1.2529x over previous
import functools

import jax
import jax.numpy as jnp
from jax.experimental import pallas as pl
from jax.experimental.pallas import tpu as pltpu


def _fused_kernel(n_steps, blk, x_ref, w1_ref, w2_ref, o_ref):
    n = pl.program_id(0)

    pooled = jnp.max(x_ref[...], axis=-1)
    row = pl.multiple_of(n * blk, 8)
    o_ref[pl.ds(row, blk), :] = pooled

    @pl.when(n == n_steps - 1)
    def _fc():
        p = o_ref[...]
        h = jax.lax.dot_general(
            p, w1_ref[...], (((1,), (1,)), ((), ())),
            preferred_element_type=jnp.float32)
        h = jnp.maximum(h, 0.0)
        out = jax.lax.dot_general(
            h, w2_ref[...], (((1,), (1,)), ((), ())),
            preferred_element_type=jnp.float32)
        o_ref[...] = jax.nn.sigmoid(out)


@functools.partial(jax.jit, static_argnames=("blk",))
def _channel_attention(x, w1, w2, blk=8):
    N, C, H, W = x.shape
    HW = H * W
    n_steps = N // blk
    x_flat = x.reshape(N, C, HW)

    out = pl.pallas_call(
        functools.partial(_fused_kernel, n_steps, blk),
        out_shape=jax.ShapeDtypeStruct((N, C), jnp.float32),
        grid=(n_steps,),
        in_specs=[
            pl.BlockSpec((blk, C, HW), lambda n: (n, 0, 0)),
            pl.BlockSpec(w1.shape, lambda n: (0, 0)),
            pl.BlockSpec(w2.shape, lambda n: (0, 0)),
        ],
        out_specs=pl.BlockSpec((N, C), lambda n: (0, 0)),
        compiler_params=pltpu.CompilerParams(
            dimension_semantics=("arbitrary",),
            vmem_limit_bytes=60 * 1024 * 1024,
        ),
    )(x_flat, w1, w2)

    return out.reshape(N, C, 1, 1)


def kernel(x, w1, w2):
    return _channel_attention(x, w1, w2)
